# Initial kernel scaffold; baseline (speedup 1.0000x reference)
import functools

import jax
import jax.numpy as jnp
from jax.experimental import pallas as pl
from jax.experimental.pallas import tpu as pltpu

_N = 25000
_D = 128
_NPAD = 25088
_RBLK = 3584


def _dense_body(wb_ref, feat_ref, g0_ref, g1_ref, wfc_ref, bfc_ref, wenc_ref,
                benc_ref, wdecp_ref, bdecp_ref, wdeca_ref, bdeca_ref, avec_ref,
                wh_ref, m1_ref, m2_ref, ssrc1_ref, ssrc2_ref, sdst1_ref,
                sdst2_ref):
    f = feat_ref[...]
    wh = jnp.dot(f, wfc_ref[...], preferred_element_type=jnp.float32) + bfc_ref[...]
    we = jnp.dot(f, wenc_ref[...], preferred_element_type=jnp.float32) + benc_ref[...]
    wb0 = wb_ref[0]
    wb1 = wb_ref[1]
    l0 = we * wb0 + g0_ref[...]
    l1 = we * wb1 + g1_ref[...]
    msg = (l1 > l0).astype(jnp.float32)
    m1 = jnp.dot(msg, wdecp_ref[...], preferred_element_type=jnp.float32) + bdecp_ref[...]
    m2 = jnp.dot(msg, wdeca_ref[...], preferred_element_type=jnp.float32) + bdeca_ref[...]
    wh_ref[...] = wh
    m1_ref[...] = m1
    m2_ref[...] = m2
    av = avec_ref[...]
    ssrc1_ref[...] = jnp.sum(m1 * av[0:1, :], axis=1)
    ssrc2_ref[...] = jnp.sum(m2 * av[1:2, :], axis=1)
    sdst1_ref[...] = jnp.sum(wh * av[2:3, :], axis=1)
    sdst2_ref[...] = jnp.sum(wh * av[3:4, :], axis=1)


def _dense_stage(feat_pad, g0_pad, g1_pad, wb, wfc_t, bfc, wenc_t, benc,
                 wdecp_t, bdecp, wdeca_t, bdeca, avec):
    nblk = _NPAD // _RBLK
    row_spec = pl.BlockSpec((_RBLK, _D), lambda i: (i, 0))
    w_spec = pl.BlockSpec((_D, _D), lambda i: (0, 0))
    b_spec = pl.BlockSpec((1, _D), lambda i: (0, 0))
    s_spec = pl.BlockSpec((_RBLK,), lambda i: (i,))
    return pl.pallas_call(
        _dense_body,
        grid=(nblk,),
        in_specs=[
            pl.BlockSpec(memory_space=pltpu.SMEM),
            row_spec, row_spec, row_spec,
            w_spec, b_spec, w_spec, b_spec,
            w_spec, b_spec, w_spec, b_spec,
            pl.BlockSpec((4, _D), lambda i: (0, 0)),
        ],
        out_specs=[row_spec, row_spec, row_spec, s_spec, s_spec, s_spec, s_spec],
        out_shape=[
            jax.ShapeDtypeStruct((_NPAD, _D), jnp.float32),
            jax.ShapeDtypeStruct((_NPAD, _D), jnp.float32),
            jax.ShapeDtypeStruct((_NPAD, _D), jnp.float32),
            jax.ShapeDtypeStruct((_NPAD,), jnp.float32),
            jax.ShapeDtypeStruct((_NPAD,), jnp.float32),
            jax.ShapeDtypeStruct((_NPAD,), jnp.float32),
            jax.ShapeDtypeStruct((_NPAD,), jnp.float32),
        ],
    )(wb, feat_pad, g0_pad, g1_pad, wfc_t, bfc, wenc_t, benc, wdecp_t, bdecp,
      wdeca_t, bdeca, avec)


def _rel_softmax_agg(ssrc, sdst, rows, edge_index, shift):
    src = edge_index[0]
    dst = edge_index[1]
    z = ssrc[src] + sdst[dst]
    e = jnp.where(z >= 0, z, 0.2 * z)
    ex = jnp.exp(e - shift)
    denom = jax.ops.segment_sum(ex, dst, num_segments=_N)
    num = jax.ops.segment_sum(ex[:, None] * rows[src], dst, num_segments=_N)
    inv = jnp.where(denom > 0, 1.0 / denom, 0.0)
    return num * inv[:, None]


def kernel(feat_P, feat_A, edge_index_p2p, edge_index_p2a, edge_index_a2p,
           edge_index_a2a, Wfc_P, bfc_P, Wfc_A, bfc_A, Wenc_P, benc_P, Wenc_A,
           benc_A, Wbin, bbin, Wdec_P, bdec_P, Wdec_A, bdec_A, a_p2p, a_p2a,
           a_a2p, a_a2a):
    kg1, kg2 = jax.random.split(jax.random.key(42))
    pad = _NPAD - _N

    def gum(key):
        u = jax.random.uniform(key, (_N, _D, 2), minval=1e-6, maxval=1.0 - 1e-6)
        g = -jnp.log(-jnp.log(u))
        g = g + bbin
        g0 = jnp.pad(g[:, :, 0], ((0, pad), (0, 0)))
        g1 = jnp.pad(g[:, :, 1], ((0, pad), (0, 0)))
        return g0, g1

    g0p, g1p = gum(kg1)
    g0a, g1a = gum(kg2)

    wb = Wbin[:, 0]
    fP = jnp.pad(feat_P, ((0, pad), (0, 0)))
    fA = jnp.pad(feat_A, ((0, pad), (0, 0)))

    avec_P = jnp.concatenate(
        [a_p2p[:, :_D], a_p2a[:, :_D], a_p2p[:, _D:], a_a2p[:, _D:]], axis=0)
    avec_A = jnp.concatenate(
        [a_a2p[:, :_D], a_a2a[:, :_D], a_p2a[:, _D:], a_a2a[:, _D:]], axis=0)

    whp, m_p2p, m_p2a, ss_p2p, ss_p2a, sd_p2p, sd_a2p = _dense_stage(
        fP, g0p, g1p, wb, Wfc_P.T, bfc_P[None, :], Wenc_P.T, benc_P[None, :],
        Wdec_P.T, bdec_P[None, :], Wdec_A.T, bdec_A[None, :], avec_P)
    wha, m_a2p, m_a2a, ss_a2p, ss_a2a, sd_p2a, sd_a2a = _dense_stage(
        fA, g0a, g1a, wb, Wfc_A.T, bfc_A[None, :], Wenc_A.T, benc_A[None, :],
        Wdec_P.T, bdec_P[None, :], Wdec_A.T, bdec_A[None, :], avec_A)

    whp = whp[:_N]
    wha = wha[:_N]

    def shift(ss, sd):
        zmax = jnp.max(ss[:_N]) + jnp.max(sd[:_N])
        return jnp.where(zmax >= 0, zmax, 0.2 * zmax)

    h_P = (whp
           + _rel_softmax_agg(ss_p2p[:_N], sd_p2p[:_N], m_p2p[:_N],
                              edge_index_p2p, shift(ss_p2p, sd_p2p))
           + _rel_softmax_agg(ss_a2p[:_N], sd_a2p[:_N], m_a2p[:_N],
                              edge_index_a2p, shift(ss_a2p, sd_a2p)))
    h_A = (wha
           + _rel_softmax_agg(ss_p2a[:_N], sd_p2a[:_N], m_p2a[:_N],
                              edge_index_p2a, shift(ss_p2a, sd_p2a))
           + _rel_softmax_agg(ss_a2a[:_N], sd_a2a[:_N], m_a2a[:_N],
                              edge_index_a2a, shift(ss_a2a, sd_a2a)))
    return (jax.nn.relu(h_P), jax.nn.relu(h_A))


# speedup vs baseline: 1.1445x; 1.1445x over previous
import functools

import jax
import jax.numpy as jnp
from jax.experimental import pallas as pl
from jax.experimental.pallas import tpu as pltpu

_N = 25000
_D = 128
_NPAD = 25600
_RBLK = 5120


def _dense_body(wb_ref, feat_ref, g0_ref, g1_ref, wfc_ref, bfc_ref, wenc_ref,
                benc_ref, wdecp_ref, bdecp_ref, wdeca_ref, bdeca_ref, avec_ref,
                wh_ref, m1_ref, m2_ref, ssrc1_ref, ssrc2_ref, sdst1_ref,
                sdst2_ref):
    f = feat_ref[...]
    wh = jnp.dot(f, wfc_ref[...], preferred_element_type=jnp.float32) + bfc_ref[...]
    we = jnp.dot(f, wenc_ref[...], preferred_element_type=jnp.float32) + benc_ref[...]
    wb0 = wb_ref[0]
    wb1 = wb_ref[1]
    l0 = we * wb0 + g0_ref[...]
    l1 = we * wb1 + g1_ref[...]
    msg = (l1 > l0).astype(jnp.float32)
    m1 = jnp.dot(msg, wdecp_ref[...], preferred_element_type=jnp.float32) + bdecp_ref[...]
    m2 = jnp.dot(msg, wdeca_ref[...], preferred_element_type=jnp.float32) + bdeca_ref[...]
    wh_ref[...] = wh
    m1_ref[...] = m1
    m2_ref[...] = m2
    av = avec_ref[...]
    r = _RBLK // _D
    ssrc1_ref[...] = jnp.sum(m1 * av[0:1, :], axis=1).reshape(r, _D)
    ssrc2_ref[...] = jnp.sum(m2 * av[1:2, :], axis=1).reshape(r, _D)
    sdst1_ref[...] = jnp.sum(wh * av[2:3, :], axis=1).reshape(r, _D)
    sdst2_ref[...] = jnp.sum(wh * av[3:4, :], axis=1).reshape(r, _D)


def _dense_stage(feat_pad, g0_pad, g1_pad, wb, wfc_t, bfc, wenc_t, benc,
                 wdecp_t, bdecp, wdeca_t, bdeca, avec):
    nblk = _NPAD // _RBLK
    row_spec = pl.BlockSpec((_RBLK, _D), lambda i: (i, 0))
    w_spec = pl.BlockSpec((_D, _D), lambda i: (0, 0))
    b_spec = pl.BlockSpec((1, _D), lambda i: (0, 0))
    s_spec = pl.BlockSpec((_RBLK // _D, _D), lambda i: (i, 0))
    return pl.pallas_call(
        _dense_body,
        grid=(nblk,),
        in_specs=[
            pl.BlockSpec(memory_space=pltpu.SMEM),
            row_spec, row_spec, row_spec,
            w_spec, b_spec, w_spec, b_spec,
            w_spec, b_spec, w_spec, b_spec,
            pl.BlockSpec((4, _D), lambda i: (0, 0)),
        ],
        out_specs=[row_spec, row_spec, row_spec, s_spec, s_spec, s_spec, s_spec],
        out_shape=[
            jax.ShapeDtypeStruct((_NPAD, _D), jnp.float32),
            jax.ShapeDtypeStruct((_NPAD, _D), jnp.float32),
            jax.ShapeDtypeStruct((_NPAD, _D), jnp.float32),
            jax.ShapeDtypeStruct((_NPAD // _D, _D), jnp.float32),
            jax.ShapeDtypeStruct((_NPAD // _D, _D), jnp.float32),
            jax.ShapeDtypeStruct((_NPAD // _D, _D), jnp.float32),
            jax.ShapeDtypeStruct((_NPAD // _D, _D), jnp.float32),
        ],
    )(wb, feat_pad, g0_pad, g1_pad, wfc_t, bfc, wenc_t, benc, wdecp_t, bdecp,
      wdeca_t, bdeca, avec)


def _rel_softmax_agg(ssrc, sdst, rows, edge_index, shift):
    src = edge_index[0]
    dst = edge_index[1]
    z = ssrc[src] + sdst[dst]
    e = jnp.where(z >= 0, z, 0.2 * z)
    ex = jnp.exp(e - shift)
    denom = jax.ops.segment_sum(ex, dst, num_segments=_N)
    num = jax.ops.segment_sum(ex[:, None] * rows[src], dst, num_segments=_N)
    inv = jnp.where(denom > 0, 1.0 / denom, 0.0)
    return num * inv[:, None]


def kernel(feat_P, feat_A, edge_index_p2p, edge_index_p2a, edge_index_a2p,
           edge_index_a2a, Wfc_P, bfc_P, Wfc_A, bfc_A, Wenc_P, benc_P, Wenc_A,
           benc_A, Wbin, bbin, Wdec_P, bdec_P, Wdec_A, bdec_A, a_p2p, a_p2a,
           a_a2p, a_a2a):
    kg1, kg2 = jax.random.split(jax.random.key(42))
    pad = _NPAD - _N

    def gum(key):
        u = jax.random.uniform(key, (_N, _D, 2), minval=1e-6, maxval=1.0 - 1e-6)
        g = -jnp.log(-jnp.log(u))
        g = g + bbin
        g0 = jnp.pad(g[:, :, 0], ((0, pad), (0, 0)))
        g1 = jnp.pad(g[:, :, 1], ((0, pad), (0, 0)))
        return g0, g1

    g0p, g1p = gum(kg1)
    g0a, g1a = gum(kg2)

    wb = Wbin[:, 0]
    fP = jnp.pad(feat_P, ((0, pad), (0, 0)))
    fA = jnp.pad(feat_A, ((0, pad), (0, 0)))

    avec_P = jnp.concatenate(
        [a_p2p[:, :_D], a_p2a[:, :_D], a_p2p[:, _D:], a_a2p[:, _D:]], axis=0)
    avec_A = jnp.concatenate(
        [a_a2p[:, :_D], a_a2a[:, :_D], a_p2a[:, _D:], a_a2a[:, _D:]], axis=0)

    whp, m_p2p, m_p2a, ss_p2p, ss_p2a, sd_p2p, sd_a2p = _dense_stage(
        fP, g0p, g1p, wb, Wfc_P.T, bfc_P[None, :], Wenc_P.T, benc_P[None, :],
        Wdec_P.T, bdec_P[None, :], Wdec_A.T, bdec_A[None, :], avec_P)
    wha, m_a2p, m_a2a, ss_a2p, ss_a2a, sd_p2a, sd_a2a = _dense_stage(
        fA, g0a, g1a, wb, Wfc_A.T, bfc_A[None, :], Wenc_A.T, benc_A[None, :],
        Wdec_P.T, bdec_P[None, :], Wdec_A.T, bdec_A[None, :], avec_A)
    (ss_p2p, ss_p2a, sd_p2p, sd_a2p, ss_a2p, ss_a2a, sd_p2a, sd_a2a) = (
        x.reshape(_NPAD) for x in
        (ss_p2p, ss_p2a, sd_p2p, sd_a2p, ss_a2p, ss_a2a, sd_p2a, sd_a2a))

    whp = whp[:_N]
    wha = wha[:_N]

    def shift(ss, sd):
        zmax = jnp.max(ss[:_N]) + jnp.max(sd[:_N])
        return jnp.where(zmax >= 0, zmax, 0.2 * zmax)

    h_P = (whp
           + _rel_softmax_agg(ss_p2p[:_N], sd_p2p[:_N], m_p2p[:_N],
                              edge_index_p2p, shift(ss_p2p, sd_p2p))
           + _rel_softmax_agg(ss_a2p[:_N], sd_a2p[:_N], m_a2p[:_N],
                              edge_index_a2p, shift(ss_a2p, sd_a2p)))
    h_A = (wha
           + _rel_softmax_agg(ss_p2a[:_N], sd_p2a[:_N], m_p2a[:_N],
                              edge_index_p2a, shift(ss_p2a, sd_p2a))
           + _rel_softmax_agg(ss_a2a[:_N], sd_a2a[:_N], m_a2a[:_N],
                              edge_index_a2a, shift(ss_a2a, sd_a2a)))
    return (jax.nn.relu(h_P), jax.nn.relu(h_A))


# speedup vs baseline: 7.5673x; 6.6119x over previous
import functools

import jax
import jax.numpy as jnp
from jax import lax
from jax.experimental import pallas as pl
from jax.experimental.pallas import tpu as pltpu
from jax.experimental.pallas import tpu_sc as plsc

_N = 25000
_D = 128
_NPAD = 25600
_RBLK = 5120

_E = 150000
_QS = 6400
_TR = 400
_SUB = 2048
_NSUB = 5
_ECHUNK = _SUB * _NSUB
_EPAD = _ECHUNK * 16
_GB = 32


def _dense_body(wb_ref, feat_ref, g0_ref, g1_ref, wfc_ref, bfc_ref, wenc_ref,
                benc_ref, wdecp_ref, bdecp_ref, wdeca_ref, bdeca_ref, avec_ref,
                wh_ref, m1_ref, m2_ref, ssrc1_ref, ssrc2_ref, sdst1_ref,
                sdst2_ref):
    f = feat_ref[...]
    wh = jnp.dot(f, wfc_ref[...], preferred_element_type=jnp.float32) + bfc_ref[...]
    we = jnp.dot(f, wenc_ref[...], preferred_element_type=jnp.float32) + benc_ref[...]
    l0 = we * wb_ref[0] + g0_ref[...]
    l1 = we * wb_ref[1] + g1_ref[...]
    msg = (l1 > l0).astype(jnp.float32)
    m1 = jnp.dot(msg, wdecp_ref[...], preferred_element_type=jnp.float32) + bdecp_ref[...]
    m2 = jnp.dot(msg, wdeca_ref[...], preferred_element_type=jnp.float32) + bdeca_ref[...]
    wh_ref[...] = wh
    m1_ref[...] = m1
    m2_ref[...] = m2
    av = avec_ref[...]
    r = _RBLK // _D
    ssrc1_ref[...] = jnp.sum(m1 * av[0:1, :], axis=1).reshape(r, _D)
    ssrc2_ref[...] = jnp.sum(m2 * av[1:2, :], axis=1).reshape(r, _D)
    sdst1_ref[...] = jnp.sum(wh * av[2:3, :], axis=1).reshape(r, _D)
    sdst2_ref[...] = jnp.sum(wh * av[3:4, :], axis=1).reshape(r, _D)


def _dense_stage(feat_pad, g0_pad, g1_pad, wb, wfc_t, bfc, wenc_t, benc,
                 wdecp_t, bdecp, wdeca_t, bdeca, avec):
    nblk = _NPAD // _RBLK
    row_spec = pl.BlockSpec((_RBLK, _D), lambda i: (i, 0))
    w_spec = pl.BlockSpec((_D, _D), lambda i: (0, 0))
    b_spec = pl.BlockSpec((1, _D), lambda i: (0, 0))
    s_spec = pl.BlockSpec((_RBLK // _D, _D), lambda i: (i, 0))
    return pl.pallas_call(
        _dense_body,
        grid=(nblk,),
        in_specs=[
            pl.BlockSpec(memory_space=pltpu.SMEM),
            row_spec, row_spec, row_spec,
            w_spec, b_spec, w_spec, b_spec,
            w_spec, b_spec, w_spec, b_spec,
            pl.BlockSpec((4, _D), lambda i: (0, 0)),
        ],
        out_specs=[row_spec, row_spec, row_spec, s_spec, s_spec, s_spec, s_spec],
        out_shape=[
            jax.ShapeDtypeStruct((_NPAD, _D), jnp.float32),
            jax.ShapeDtypeStruct((_NPAD, _D), jnp.float32),
            jax.ShapeDtypeStruct((_NPAD, _D), jnp.float32),
            jax.ShapeDtypeStruct((_NPAD // _D, _D), jnp.float32),
            jax.ShapeDtypeStruct((_NPAD // _D, _D), jnp.float32),
            jax.ShapeDtypeStruct((_NPAD // _D, _D), jnp.float32),
            jax.ShapeDtypeStruct((_NPAD // _D, _D), jnp.float32),
        ],
    )(wb, feat_pad, g0_pad, g1_pad, wfc_t, bfc, wenc_t, benc, wdecp_t, bdecp,
      wdeca_t, bdeca, avec)


def _combine_body(wh_ref, r1_ref, r2_ref, out_ref):
    out_ref[...] = jnp.maximum(wh_ref[...] + r1_ref[...] + r2_ref[...], 0.0)


def _combine_stage(wh, r1, r2):
    row_spec = pl.BlockSpec((_RBLK, _D), lambda i: (i, 0))
    return pl.pallas_call(
        _combine_body,
        grid=(_NPAD // _RBLK,),
        in_specs=[row_spec, row_spec, row_spec],
        out_specs=row_spec,
        out_shape=jax.ShapeDtypeStruct((_NPAD, _D), jnp.float32),
    )(wh, r1, r2)


def _sc_rel_body(qbase, src_hbm, dst_hbm, ssrc_hbm, sdst_hbm, m_hbm, shift_hbm,
                 out_hbm, ssrc_tab, sdst_tab, srcb, dstb, csrc, cdst, cex,
                 csrc_b, cdst_b, rows_v, den2d, rowiota, m16, outb,
                 acc_sh, den_sh, sem):
    c = lax.axis_index("c")
    s = lax.axis_index("s")
    base = qbase + c * _QS

    zf = jnp.zeros((16,), jnp.float32)
    zi = jnp.zeros((16,), jnp.int32)

    def _zrow(r, _):
        for k in range(8):
            outb[r, pl.ds(k * 16, 16)] = zf
        return 0
    lax.fori_loop(0, 80, _zrow, 0)

    def _zden(r, _):
        for k in range(8):
            den2d[r, pl.ds(k * 16, 16)] = zf
        return 0
    lax.fori_loop(0, 64, _zden, 0)

    def _zc(i, _):
        csrc[pl.ds(i * 16, 16)] = zi
        cdst[pl.ds(i * 16, 16)] = zi
        return 0
    lax.fori_loop(0, (_SUB + 16) // 16, _zc, 0)

    for k in range(4):
        rowiota[pl.ds(k * 16, 16)] = lax.iota(jnp.int32, 16) + (k * 16)

    srow = pl.multiple_of(s * _TR, 16)
    for j in range(_TR // 80):
        pltpu.sync_copy(outb, acc_sh.at[pl.ds(srow + j * 80, 80)])

    @pl.when(s == 0)
    def _():
        pltpu.sync_copy(den2d, den_sh)

    pltpu.sync_copy(ssrc_hbm, ssrc_tab)
    pltpu.sync_copy(sdst_hbm.at[pl.ds(pl.multiple_of(base, 128), _QS)],
                    sdst_tab)
    pltpu.sync_copy(shift_hbm, m16)
    plsc.subcore_barrier()
    mv = m16[...]

    for sub in range(_NSUB):
        ebase = pl.multiple_of(s * _ECHUNK, 128) + sub * _SUB
        pltpu.sync_copy(src_hbm.at[pl.ds(ebase, _SUB)], srcb)
        pltpu.sync_copy(dst_hbm.at[pl.ds(ebase, _SUB)], dstb)

        def _scan(i, off):
            dv = dstb[pl.ds(i * 16, 16)]
            sv = srcb[pl.ds(i * 16, 16)]
            inh = (dv >= base) & (dv < base + _QS)
            dloc = jnp.where(inh, dv - base, 0)
            es = plsc.load_gather(ssrc_tab, [sv])
            ed = plsc.load_gather(sdst_tab, [dloc])
            z = es + ed
            e = jnp.where(z >= 0, z, 0.2 * z)
            ex = jnp.where(inh, jnp.exp(e - mv), 0.0)
            plsc.store_compressed(csrc.at[pl.ds(off, 16)], sv, mask=inh)
            plsc.store_compressed(cdst.at[pl.ds(off, 16)], dloc, mask=inh)
            plsc.store_compressed(cex.at[pl.ds(off, 16)], ex, mask=inh)
            plsc.addupdate_scatter(den2d, [dloc // 128, dloc % 128], ex)
            return off + jnp.sum(inh.astype(jnp.int32))

        off = lax.fori_loop(0, _SUB // 16, _scan, jnp.int32(0))

        def _batch(b, _):
            bb = pl.multiple_of(b * _GB, 16)
            for k in range(_GB // 16):
                csrc_b[pl.ds(k * 16, 16)] = csrc[pl.ds(bb + k * 16, 16)]
                cdst_b[pl.ds(k * 16, 16)] = cdst[pl.ds(bb + k * 16, 16)]
            pltpu.async_copy(m_hbm.at[csrc_b], rows_v, sem).wait()
            hi = off - b * _GB
            for kk in range(_GB // 16):
                exv = cex[pl.ds(bb + kk * 16, 16)]
                lane = lax.iota(jnp.int32, 16) + kk * 16
                exv = jnp.where(lane < hi, exv, 0.0)
                for l in range(16):
                    ab = jnp.full((16,), exv[l], jnp.float32)
                    j = kk * 16 + l
                    for k in range(8):
                        rows_v[j, pl.ds(k * 16, 16)] = (
                            rows_v[j, pl.ds(k * 16, 16)] * ab)
            pltpu.sync_copy(rows_v, acc_sh.at[cdst_b], add=True)
            return 0

        lax.fori_loop(0, (off + _GB - 1) // _GB, _batch, 0)

    pltpu.sync_copy(den2d, den_sh.at[rowiota], add=True)
    plsc.subcore_barrier()
    pltpu.sync_copy(den_sh, den2d)

    def _invden(r, _):
        for k in range(8):
            v = den2d[r, pl.ds(k * 16, 16)]
            den2d[r, pl.ds(k * 16, 16)] = jnp.where(v > 0, 1.0 / v, 0.0)
        return 0
    lax.fori_loop(0, 64, _invden, 0)

    for j in range(_TR // 80):
        g0 = srow + j * 80
        pltpu.sync_copy(acc_sh.at[pl.ds(g0, 80)], outb)

        def _ngrp(t, _):
            pv = lax.iota(jnp.int32, 16) + (g0 + t * 16)
            iv16 = plsc.load_gather(den2d, [pv // 128, pv % 128])
            for l in range(16):
                ivb = jnp.full((16,), iv16[l], jnp.float32)
                r = t * 16 + l
                for k in range(8):
                    outb[r, pl.ds(k * 16, 16)] = (
                        outb[r, pl.ds(k * 16, 16)] * ivb)
            return 0
        lax.fori_loop(0, 5, _ngrp, 0)
        out_off = pl.multiple_of(c * _QS, 128) + g0
        pltpu.sync_copy(outb, out_hbm.at[pl.ds(out_off, 80)])


def _sc_relation(edge_index, ssrc, sdst, m_rows, shift):
    src = jnp.pad(edge_index[0], (0, _EPAD - _E))
    dst = jnp.pad(edge_index[1], (0, _EPAD - _E), constant_values=-1)
    shift16 = jnp.full((16,), shift, jnp.float32)

    scratch = [
        pltpu.VMEM((_NPAD,), jnp.float32),
        pltpu.VMEM((_QS,), jnp.float32),
        pltpu.VMEM((_SUB,), jnp.int32),
        pltpu.VMEM((_SUB,), jnp.int32),
        pltpu.VMEM((_SUB + 16,), jnp.int32),
        pltpu.VMEM((_SUB + 16,), jnp.int32),
        pltpu.VMEM((_SUB + 16,), jnp.float32),
        pltpu.VMEM((_GB,), jnp.int32),
        pltpu.VMEM((_GB,), jnp.int32),
        pltpu.VMEM((_GB, _D), jnp.float32),
        pltpu.VMEM((64, 128), jnp.float32),
        pltpu.VMEM((64,), jnp.int32),
        pltpu.VMEM((16,), jnp.float32),
        pltpu.VMEM((80, _D), jnp.float32),
        pltpu.VMEM_SHARED((_QS, _D), jnp.float32),
        pltpu.VMEM_SHARED((64, 128), jnp.float32),
        pltpu.SemaphoreType.DMA,
    ]

    parts = []
    for q in range(2):
        fn = pl.kernel(
            functools.partial(_sc_rel_body, q * 2 * _QS),
            out_type=jax.ShapeDtypeStruct((2 * _QS, _D), jnp.float32),
            mesh=plsc.VectorSubcoreMesh(core_axis_name="c",
                                        subcore_axis_name="s",
                                        num_cores=2, num_subcores=16),
            scratch_types=scratch,
            compiler_params=pltpu.CompilerParams(needs_layout_passes=False),
        )
        parts.append(fn(src, dst, ssrc, sdst, m_rows, shift16))
    return jnp.concatenate(parts, axis=0)


def kernel(feat_P, feat_A, edge_index_p2p, edge_index_p2a, edge_index_a2p,
           edge_index_a2a, Wfc_P, bfc_P, Wfc_A, bfc_A, Wenc_P, benc_P, Wenc_A,
           benc_A, Wbin, bbin, Wdec_P, bdec_P, Wdec_A, bdec_A, a_p2p, a_p2a,
           a_a2p, a_a2a):
    kg1, kg2 = jax.random.split(jax.random.key(42))
    pad = _NPAD - _N

    def gum(key):
        u = jax.random.uniform(key, (_N, _D, 2), minval=1e-6, maxval=1.0 - 1e-6)
        g = -jnp.log(-jnp.log(u))
        g = g + bbin
        g0 = jnp.pad(g[:, :, 0], ((0, pad), (0, 0)))
        g1 = jnp.pad(g[:, :, 1], ((0, pad), (0, 0)))
        return g0, g1

    g0p, g1p = gum(kg1)
    g0a, g1a = gum(kg2)

    wb = Wbin[:, 0]
    fP = jnp.pad(feat_P, ((0, pad), (0, 0)))
    fA = jnp.pad(feat_A, ((0, pad), (0, 0)))

    avec_P = jnp.concatenate(
        [a_p2p[:, :_D], a_p2a[:, :_D], a_p2p[:, _D:], a_a2p[:, _D:]], axis=0)
    avec_A = jnp.concatenate(
        [a_a2p[:, :_D], a_a2a[:, :_D], a_p2a[:, _D:], a_a2a[:, _D:]], axis=0)

    whp, m_p2p, m_p2a, ss_p2p, ss_p2a, sd_p2p, sd_a2p = _dense_stage(
        fP, g0p, g1p, wb, Wfc_P.T, bfc_P[None, :], Wenc_P.T, benc_P[None, :],
        Wdec_P.T, bdec_P[None, :], Wdec_A.T, bdec_A[None, :], avec_P)
    wha, m_a2p, m_a2a, ss_a2p, ss_a2a, sd_p2a, sd_a2a = _dense_stage(
        fA, g0a, g1a, wb, Wfc_A.T, bfc_A[None, :], Wenc_A.T, benc_A[None, :],
        Wdec_P.T, bdec_P[None, :], Wdec_A.T, bdec_A[None, :], avec_A)
    (ss_p2p, ss_p2a, sd_p2p, sd_a2p, ss_a2p, ss_a2a, sd_p2a, sd_a2a) = (
        x.reshape(_NPAD) for x in
        (ss_p2p, ss_p2a, sd_p2p, sd_a2p, ss_a2p, ss_a2a, sd_p2a, sd_a2a))

    def shift(ss, sd):
        zmax = jnp.max(ss[:_N]) + jnp.max(sd[:_N])
        return jnp.where(zmax >= 0, zmax, 0.2 * zmax)

    r1p = _sc_relation(edge_index_p2p, ss_p2p, sd_p2p, m_p2p,
                       shift(ss_p2p, sd_p2p))
    r2p = _sc_relation(edge_index_a2p, ss_a2p, sd_a2p, m_a2p,
                       shift(ss_a2p, sd_a2p))
    r1a = _sc_relation(edge_index_p2a, ss_p2a, sd_p2a, m_p2a,
                       shift(ss_p2a, sd_p2a))
    r2a = _sc_relation(edge_index_a2a, ss_a2a, sd_a2a, m_a2a,
                       shift(ss_a2a, sd_a2a))

    h_P = _combine_stage(whp, r1p, r2p)
    h_A = _combine_stage(wha, r1a, r2a)
    return (h_P[:_N], h_A[:_N])
